# baseline (device time: 370673 ns/iter reference)
import jax
import jax.numpy as jnp
from jax import lax
from jax.experimental import pallas as pl
from jax.experimental.pallas import tpu as pltpu

W = 32
M = 1536
N = 1536
CH = M // W
H = CH // 2


def kernel(A, B):
    def body(a_ref, b_ref, out_ref, comm_cw, comm_ccw,
             cw_send, cw_recv, ccw_send, ccw_recv):
        my = lax.axis_index("i")
        left = lax.rem(my + (W - 1), W)
        right = lax.rem(my + 1, W)

        barrier = pltpu.get_barrier_semaphore()
        pl.semaphore_signal(barrier, inc=1, device_id=(left,),
                            device_id_type=pl.DeviceIdType.MESH)
        pl.semaphore_signal(barrier, inc=1, device_id=(right,),
                            device_id_type=pl.DeviceIdType.MESH)
        pl.semaphore_wait(barrier, 2)

        out_ref[...] = jnp.dot(a_ref[...], b_ref[...],
                               preferred_element_type=jnp.float32)

        prev = []

        def remote(src, dst, send_sem, recv_sem, dev):
            return pltpu.make_async_remote_copy(
                src_ref=src, dst_ref=dst, send_sem=send_sem,
                recv_sem=recv_sem, device_id=(dev,),
                device_id_type=pl.DeviceIdType.MESH,
            )

        for s in range(W - 1):
            cs = lax.rem(my - s + W, W)
            cr = lax.rem(my - s - 1 + W, W)
            cs2 = lax.rem(my + s, W)
            cr2 = lax.rem(my + s + 1, W)
            d_cw = remote(out_ref.at[pl.ds(cs * CH, H), :], comm_cw.at[s],
                          cw_send.at[s], cw_recv.at[s], right)
            d_ccw = remote(out_ref.at[pl.ds(cs2 * CH + H, H), :],
                           comm_ccw.at[s],
                           ccw_send.at[s], ccw_recv.at[s], left)
            d_cw.start()
            d_ccw.start()
            d_cw.wait_recv()
            out_ref[pl.ds(cr * CH, H), :] = (
                out_ref[pl.ds(cr * CH, H), :] + comm_cw[s]
            )
            d_ccw.wait_recv()
            out_ref[pl.ds(cr2 * CH + H, H), :] = (
                out_ref[pl.ds(cr2 * CH + H, H), :] + comm_ccw[s]
            )
            for d in prev:
                d.wait_send()
            prev = [d_cw, d_ccw]

        for s in range(W - 1):
            ga = lax.rem(my + 1 - s + W, W)
            gb = lax.rem(my - 1 + s + W, W)
            d_cw = remote(out_ref.at[pl.ds(ga * CH, H), :],
                          out_ref.at[pl.ds(ga * CH, H), :],
                          cw_send.at[s], cw_recv.at[s],
                          right)
            d_ccw = remote(out_ref.at[pl.ds(gb * CH + H, H), :],
                           out_ref.at[pl.ds(gb * CH + H, H), :],
                           ccw_send.at[s], ccw_recv.at[s],
                           left)
            d_cw.start()
            d_ccw.start()
            d_cw.wait_recv()
            d_ccw.wait_recv()
            for d in prev:
                d.wait_send()
            prev = [d_cw, d_ccw]

        for d in prev:
            d.wait_send()

        z = out_ref[...]
        out_ref[...] = 0.5 * z * (
            1.0 + jnp.tanh(0.7978845608 * (z + 0.044715 * z * z * z))
        )

    return pl.pallas_call(
        body,
        out_shape=jax.ShapeDtypeStruct((M, N), jnp.float32),
        in_specs=[
            pl.BlockSpec(memory_space=pltpu.VMEM),
            pl.BlockSpec(memory_space=pltpu.VMEM),
        ],
        out_specs=pl.BlockSpec(memory_space=pltpu.VMEM),
        scratch_shapes=[
            pltpu.VMEM((W - 1, H, N), jnp.float32),
            pltpu.VMEM((W - 1, H, N), jnp.float32),
            pltpu.SemaphoreType.DMA(((W - 1),)),
            pltpu.SemaphoreType.DMA(((W - 1),)),
            pltpu.SemaphoreType.DMA(((W - 1),)),
            pltpu.SemaphoreType.DMA(((W - 1),)),
        ],
        compiler_params=pltpu.CompilerParams(collective_id=0),
    )(A, B)
